# baseline (speedup 1.0000x reference)
import jax
import jax.numpy as jnp
from jax.experimental import pallas as pl
from jax.experimental.pallas import tpu as pltpu

_DN = (((1,), (1,)), ((), ()))

_SCALE = 30.0 / 3.14159265358979323846
_A1 = 3.141592653589793
_A3 = -5.167712780049970
_A5 = 2.550164039877345
_A7 = -0.599264529320792
_A9 = 0.082145886611128


def _sinpi(t):
    q = jnp.round(t)
    r = t - q
    sgn = jax.lax.shift_left(q.astype(jnp.int32), 31)
    r2 = r * r
    p = _A9
    for c in (_A7, _A5, _A3, _A1):
        p = p * r2 + c
    res = jax.lax.bitcast_convert_type(p * r, jnp.int32) ^ sgn
    return jax.lax.bitcast_convert_type(res, jnp.float32)


def _f32dot(a, b, dn=None):
    if dn is None:
        return jnp.dot(a, b, preferred_element_type=jnp.float32)
    return jax.lax.dot_general(a, b, dn, preferred_element_type=jnp.float32)


def _gen_small_body(*refs):
    z = refs[0][...]

    def head(i):
        hw = jnp.maximum(_f32dot(z, refs[i][...]) + refs[i + 1][...], 0.0)
        return _f32dot(hw, refs[i + 2][...]) + refs[i + 3][...]

    n_in = 1 + 4 * 7
    outs = refs[n_in:]
    for j in range(7):
        s = 1.0 if j in (1, 6) else _SCALE
        outs[j][...] = head(1 + 4 * j) * s


def _gen_small(z, hp):
    B, Z = z.shape
    row = lambda v: v.reshape(1, -1)
    args = [z]
    out_shapes = []
    for l in (0, 4):
        args += [hp[f'w_h1_{l}'], row(hp[f'w_h1b_{l}']),
                 hp[f'w_h2_{l}'], row(hp[f'w_h2b_{l}'])]
        out_shapes.append(jax.ShapeDtypeStruct((B, hp[f'w_h2_{l}'].shape[1]),
                                               jnp.float32))
    for l in range(5):
        args += [hp[f'b_h1_{l}'], row(hp[f'b_h1b_{l}']),
                 hp[f'b_h2_{l}'], row(hp[f'b_h2b_{l}'])]
        out_shapes.append(jax.ShapeDtypeStruct((B, hp[f'b_h2_{l}'].shape[1]),
                                               jnp.float32))
    return pl.pallas_call(_gen_small_body, out_shape=out_shapes)(*args)


_CHUNK = 2048
_PH = 32
_NHALF = 1024


def _mega_body(x_ref, z_ref, wh1_1, wh1b_1, wh1_2, wh1b_2, wh1_3, wh1b_3,
               w2_1, w2b_1, w2_2, w2b_2, w2_3, w2b_3,
               w0_ref, w4_ref, b0_ref, b1_ref, b2_ref, b3_ref, b4_ref,
               out_ref, H, WA, WB):
    s = pl.program_id(0)
    z = z_ref[...]

    def gen_chunk(Ws, w2blk_ref, w2bblk_ref, wh1_ref, wh1b_ref, c):
        hw = jnp.maximum(_f32dot(z, wh1_ref[...]) + wh1b_ref[...], 0.0)
        blk = w2blk_ref[...]
        bb = w2bblk_ref[...]
        pieces = []
        for k in range(_CHUNK // 256):
            v = _f32dot(hw, blk[:, 256 * k:256 * (k + 1)])
            v = (v + bb[:, 256 * k:256 * (k + 1)]) * _SCALE
            pieces.append(v[:, None, :])
        rows = jnp.concatenate(pieces, axis=1)
        Ws[:, pl.ds((_CHUNK // 256) * c, _CHUNK // 256), :] = rows

    @pl.when(s < _PH)
    def _phase0():
        gen_chunk(WA, w2_1, w2b_1, wh1_1, wh1b_1, s)
        b, p = s // 2, s % 2
        xb = x_ref[0, pl.ds(_NHALF * p, _NHALF), :]
        h0 = _sinpi(_f32dot(xb, w0_ref[0], _DN) + b0_ref[b])
        H[b, pl.ds(_NHALF * p, _NHALF), :] = h0

    @pl.when(jnp.logical_and(s >= _PH, s < 2 * _PH))
    def _phase1():
        t = s - _PH
        gen_chunk(WB, w2_2, w2b_2, wh1_2, wh1b_2, t)
        b, p = t // 2, t % 2
        hin = H[b, pl.ds(_NHALF * p, _NHALF), :]
        h1 = _sinpi(_f32dot(hin, WA[b], _DN) + b1_ref[b])
        H[b, pl.ds(_NHALF * p, _NHALF), :] = h1

    @pl.when(jnp.logical_and(s >= 2 * _PH, s < 3 * _PH))
    def _phase2():
        t = s - 2 * _PH
        gen_chunk(WA, w2_3, w2b_3, wh1_3, wh1b_3, t)
        b, p = t // 2, t % 2
        hin = H[b, pl.ds(_NHALF * p, _NHALF), :]
        h2 = _sinpi(_f32dot(hin, WB[b], _DN) + b2_ref[b])
        H[b, pl.ds(_NHALF * p, _NHALF), :] = h2

    @pl.when(s >= 3 * _PH)
    def _phase3():
        t = s - 3 * _PH
        b, p = t // 2, t % 2
        hin = H[b, pl.ds(_NHALF * p, _NHALF), :]
        h3 = _sinpi(_f32dot(hin, WA[b], _DN) + b3_ref[b])
        out_ref[0] = _f32dot(w4_ref[b], h3, _DN) + b4_ref[b]


def kernel(query_points, z_object, hyper_params):
    hp = hyper_params
    B, Z = z_object.shape
    _, N, D = query_points.shape
    H = 256
    row = lambda v: v.reshape(1, -1)
    W0, W4, b0, b1, b2, b3, b4 = _gen_small(z_object, hp)

    full = lambda shape: pl.BlockSpec(shape, lambda s: (0,) * len(shape))
    nsteps = 4 * _PH

    def stream_spec(phase):
        lo = phase * _PH
        return pl.BlockSpec((H, _CHUNK),
                            lambda s, lo=lo: (0, jnp.clip(s - lo, 0, _PH - 1)))

    def streamb_spec(phase):
        lo = phase * _PH
        return pl.BlockSpec((1, _CHUNK),
                            lambda s, lo=lo: (0, jnp.clip(s - lo, 0, _PH - 1)))

    ex0 = lambda s: (jnp.minimum(s // 2, 15), 0, 0)
    in_specs = [
        pl.BlockSpec((1, N, D), ex0),
        full((B, Z)),
    ]
    for l in (1, 2, 3):
        in_specs += [full((Z, H)), full((1, H))]
    in_specs += [stream_spec(0), streamb_spec(0),
                 stream_spec(1), streamb_spec(1),
                 stream_spec(2), streamb_spec(2)]
    in_specs += [pl.BlockSpec((1, H, D), ex0), full((B, 1, H))]
    in_specs += [full((B, 1, H))] * 4
    in_specs += [full((B, 1, 1))]

    args = [query_points, z_object]
    for l in (1, 2, 3):
        args += [hp[f'w_h1_{l}'], row(hp[f'w_h1b_{l}'])]
    stream_args = []
    for l in (1, 2, 3):
        stream_args += [hp[f'w_h2_{l}'], row(hp[f'w_h2b_{l}'])]
    args += stream_args
    args += [W0.reshape(B, H, D), W4.reshape(B, 1, H),
             b0.reshape(B, 1, H), b1.reshape(B, 1, H),
             b2.reshape(B, 1, H), b3.reshape(B, 1, H),
             b4.reshape(B, 1, 1)]

    out = pl.pallas_call(
        _mega_body,
        grid=(nsteps,),
        in_specs=in_specs,
        out_specs=pl.BlockSpec((1, 1, _NHALF),
                               lambda s: (jnp.clip(s - 3 * _PH, 0, 2 * B - 1),
                                          0, 0)),
        out_shape=jax.ShapeDtypeStruct((2 * B, 1, _NHALF), jnp.float32),
        scratch_shapes=[
            pltpu.VMEM((B, N, H), jnp.float32),
            pltpu.VMEM((B, H, H), jnp.float32),
            pltpu.VMEM((B, H, H), jnp.float32),
        ],
    )(*args)
    return out.reshape(B, N)


# speedup vs baseline: 1.0283x; 1.0283x over previous
import jax
import jax.numpy as jnp
from jax.experimental import pallas as pl
from jax.experimental.pallas import tpu as pltpu

_DN = (((1,), (1,)), ((), ()))

_SCALE = 30.0 / 3.14159265358979323846
_A1 = 3.14158291
_A3 = -5.16717838
_A5 = 2.54224131
_A7 = -0.5555504


def _sinpi(t):
    q = jnp.round(t)
    r = t - q
    sgn = jax.lax.shift_left(q.astype(jnp.int32), 31)
    r2 = r * r
    p = _A7
    for c in (_A5, _A3, _A1):
        p = p * r2 + c
    res = jax.lax.bitcast_convert_type(p * r, jnp.int32) ^ sgn
    return jax.lax.bitcast_convert_type(res, jnp.float32)


def _f32dot(a, b, dn=None):
    if dn is None:
        return jnp.dot(a, b, preferred_element_type=jnp.float32)
    return jax.lax.dot_general(a, b, dn, preferred_element_type=jnp.float32)


def _gen_small_body(*refs):
    z = refs[0][...]

    def head(i):
        hw = jnp.maximum(_f32dot(z, refs[i][...]) + refs[i + 1][...], 0.0)
        return _f32dot(hw, refs[i + 2][...]) + refs[i + 3][...]

    n_in = 1 + 4 * 7
    outs = refs[n_in:]
    for j in range(7):
        s = 1.0 if j in (1, 6) else _SCALE
        outs[j][...] = head(1 + 4 * j) * s


def _gen_small(z, hp):
    B, Z = z.shape
    row = lambda v: v.reshape(1, -1)
    args = [z]
    out_shapes = []
    for l in (0, 4):
        args += [hp[f'w_h1_{l}'], row(hp[f'w_h1b_{l}']),
                 hp[f'w_h2_{l}'], row(hp[f'w_h2b_{l}'])]
        out_shapes.append(jax.ShapeDtypeStruct((B, hp[f'w_h2_{l}'].shape[1]),
                                               jnp.float32))
    for l in range(5):
        args += [hp[f'b_h1_{l}'], row(hp[f'b_h1b_{l}']),
                 hp[f'b_h2_{l}'], row(hp[f'b_h2b_{l}'])]
        out_shapes.append(jax.ShapeDtypeStruct((B, hp[f'b_h2_{l}'].shape[1]),
                                               jnp.float32))
    return pl.pallas_call(_gen_small_body, out_shape=out_shapes)(*args)


_CHUNK = 2048
_PH = 32
_NHALF = 1024


def _mega_body(x_ref, z_ref, wh1_1, wh1b_1, wh1_2, wh1b_2, wh1_3, wh1b_3,
               w2_1, w2b_1, w2_2, w2b_2, w2_3, w2b_3,
               w0_ref, w4_ref, b0_ref, b1_ref, b2_ref, b3_ref, b4_ref,
               out_ref, H, WA, WB):
    s = pl.program_id(0)
    z = z_ref[...]

    def gen_chunk(Ws, w2blk_ref, w2bblk_ref, wh1_ref, wh1b_ref, c):
        hw = jnp.maximum(_f32dot(z, wh1_ref[...]) + wh1b_ref[...], 0.0)
        blk = w2blk_ref[...]
        bb = w2bblk_ref[...]
        pieces = []
        for k in range(_CHUNK // 256):
            v = _f32dot(hw, blk[:, 256 * k:256 * (k + 1)])
            v = (v + bb[:, 256 * k:256 * (k + 1)]) * _SCALE
            pieces.append(v[:, None, :])
        rows = jnp.concatenate(pieces, axis=1)
        Ws[:, pl.ds((_CHUNK // 256) * c, _CHUNK // 256), :] = rows

    @pl.when(s < _PH)
    def _phase0():
        gen_chunk(WA, w2_1, w2b_1, wh1_1, wh1b_1, s)
        b, p = s // 2, s % 2
        xb = x_ref[0, pl.ds(_NHALF * p, _NHALF), :]
        h0 = _sinpi(_f32dot(xb, w0_ref[0], _DN) + b0_ref[b])
        H[b, pl.ds(_NHALF * p, _NHALF), :] = h0

    @pl.when(jnp.logical_and(s >= _PH, s < 2 * _PH))
    def _phase1():
        t = s - _PH
        gen_chunk(WB, w2_2, w2b_2, wh1_2, wh1b_2, t)
        b, p = t // 2, t % 2
        hin = H[b, pl.ds(_NHALF * p, _NHALF), :]
        h1 = _sinpi(_f32dot(hin, WA[b], _DN) + b1_ref[b])
        H[b, pl.ds(_NHALF * p, _NHALF), :] = h1

    @pl.when(jnp.logical_and(s >= 2 * _PH, s < 3 * _PH))
    def _phase2():
        t = s - 2 * _PH
        gen_chunk(WA, w2_3, w2b_3, wh1_3, wh1b_3, t)
        b, p = t // 2, t % 2
        hin = H[b, pl.ds(_NHALF * p, _NHALF), :]
        h2 = _sinpi(_f32dot(hin, WB[b], _DN) + b2_ref[b])
        H[b, pl.ds(_NHALF * p, _NHALF), :] = h2

    @pl.when(s >= 3 * _PH)
    def _phase3():
        t = s - 3 * _PH
        b, p = t // 2, t % 2
        hin = H[b, pl.ds(_NHALF * p, _NHALF), :]
        h3 = _sinpi(_f32dot(hin, WA[b], _DN) + b3_ref[b])
        out_ref[0] = _f32dot(w4_ref[b], h3, _DN) + b4_ref[b]


def kernel(query_points, z_object, hyper_params):
    hp = hyper_params
    B, Z = z_object.shape
    _, N, D = query_points.shape
    H = 256
    row = lambda v: v.reshape(1, -1)
    W0, W4, b0, b1, b2, b3, b4 = _gen_small(z_object, hp)

    full = lambda shape: pl.BlockSpec(shape, lambda s: (0,) * len(shape))
    nsteps = 4 * _PH

    def stream_spec(phase):
        lo = phase * _PH
        return pl.BlockSpec((H, _CHUNK),
                            lambda s, lo=lo: (0, jnp.clip(s - lo, 0, _PH - 1)))

    def streamb_spec(phase):
        lo = phase * _PH
        return pl.BlockSpec((1, _CHUNK),
                            lambda s, lo=lo: (0, jnp.clip(s - lo, 0, _PH - 1)))

    ex0 = lambda s: (jnp.minimum(s // 2, 15), 0, 0)
    in_specs = [
        pl.BlockSpec((1, N, D), ex0),
        full((B, Z)),
    ]
    for l in (1, 2, 3):
        in_specs += [full((Z, H)), full((1, H))]
    in_specs += [stream_spec(0), streamb_spec(0),
                 stream_spec(1), streamb_spec(1),
                 stream_spec(2), streamb_spec(2)]
    in_specs += [pl.BlockSpec((1, H, D), ex0), full((B, 1, H))]
    in_specs += [full((B, 1, H))] * 4
    in_specs += [full((B, 1, 1))]

    args = [query_points, z_object]
    for l in (1, 2, 3):
        args += [hp[f'w_h1_{l}'], row(hp[f'w_h1b_{l}'])]
    stream_args = []
    for l in (1, 2, 3):
        stream_args += [hp[f'w_h2_{l}'], row(hp[f'w_h2b_{l}'])]
    args += stream_args
    args += [W0.reshape(B, H, D), W4.reshape(B, 1, H),
             b0.reshape(B, 1, H), b1.reshape(B, 1, H),
             b2.reshape(B, 1, H), b3.reshape(B, 1, H),
             b4.reshape(B, 1, 1)]

    out = pl.pallas_call(
        _mega_body,
        grid=(nsteps,),
        in_specs=in_specs,
        out_specs=pl.BlockSpec((1, 1, _NHALF),
                               lambda s: (jnp.clip(s - 3 * _PH, 0, 2 * B - 1),
                                          0, 0)),
        out_shape=jax.ShapeDtypeStruct((2 * B, 1, _NHALF), jnp.float32),
        scratch_shapes=[
            pltpu.VMEM((B, N, H), jnp.float32),
            pltpu.VMEM((B, H, H), jnp.float32),
            pltpu.VMEM((B, H, H), jnp.float32),
        ],
    )(*args)
    return out.reshape(B, N)


# speedup vs baseline: 1.2264x; 1.1926x over previous
import jax
import jax.numpy as jnp
from jax.experimental import pallas as pl
from jax.experimental.pallas import tpu as pltpu

_DN = (((1,), (1,)), ((), ()))

_SCALE = 30.0 / 3.14159265358979323846
_A1 = 3.14158291
_A3 = -5.16717838
_A5 = 2.54224131
_A7 = -0.5555504


def _sinpi(t):
    q = jnp.round(t)
    r = t - q
    sgn = jax.lax.shift_left(q.astype(jnp.int32), 31)
    r2 = r * r
    p = _A7
    for c in (_A5, _A3, _A1):
        p = p * r2 + c
    res = jax.lax.bitcast_convert_type(p * r, jnp.int32) ^ sgn
    return jax.lax.bitcast_convert_type(res, jnp.float32)


def _f32dot(a, b, dn=None):
    if dn is None:
        return jnp.dot(a, b, preferred_element_type=jnp.float32)
    return jax.lax.dot_general(a, b, dn, preferred_element_type=jnp.float32)


def _gen_small_body(*refs):
    z = refs[0][...]

    def head(i):
        hw = jnp.maximum(_f32dot(z, refs[i][...]) + refs[i + 1][...], 0.0)
        return _f32dot(hw, refs[i + 2][...]) + refs[i + 3][...]

    n_in = 1 + 4 * 7
    outs = refs[n_in:]
    for j in range(7):
        s = 1.0 if j in (1, 6) else _SCALE
        outs[j][...] = head(1 + 4 * j) * s


def _gen_small(z, hp):
    B, Z = z.shape
    row = lambda v: v.reshape(1, -1)
    args = [z]
    out_shapes = []
    for l in (0, 4):
        args += [hp[f'w_h1_{l}'], row(hp[f'w_h1b_{l}']),
                 hp[f'w_h2_{l}'], row(hp[f'w_h2b_{l}'])]
        out_shapes.append(jax.ShapeDtypeStruct((B, hp[f'w_h2_{l}'].shape[1]),
                                               jnp.float32))
    for l in range(5):
        args += [hp[f'b_h1_{l}'], row(hp[f'b_h1b_{l}']),
                 hp[f'b_h2_{l}'], row(hp[f'b_h2b_{l}'])]
        out_shapes.append(jax.ShapeDtypeStruct((B, hp[f'b_h2_{l}'].shape[1]),
                                               jnp.float32))
    return pl.pallas_call(_gen_small_body, out_shape=out_shapes)(*args)


_CHUNK = 2048
_CHUNK0 = 4096
_PH = 32
_NHALF = 1024


def _mega_body(x_ref, z_ref, wh1_1, wh1b_1, wh1_2, wh1b_2, wh1_3, wh1b_3,
               w2_1, w2b_1, w2_2, w2b_2, w2_3, w2b_3,
               w0_ref, w4_ref, b0_ref, b1_ref, b2_ref, b3_ref, b4_ref,
               out_ref, H, WA, WB):
    s = pl.program_id(0)
    z = z_ref[...]

    def gen_chunk(Ws, w2blk_ref, w2bblk_ref, wh1_ref, wh1b_ref, c, cols):
        hw = jnp.maximum(_f32dot(z, wh1_ref[...]) + wh1b_ref[...], 0.0)
        blk = w2blk_ref[...]
        bb = w2bblk_ref[...]
        pieces = []
        for k in range(cols // 256):
            v = _f32dot(hw, blk[:, 256 * k:256 * (k + 1)])
            v = (v + bb[:, 256 * k:256 * (k + 1)]) * _SCALE
            pieces.append(v[:, None, :])
        rows = jnp.concatenate(pieces, axis=1)
        Ws[:, pl.ds((cols // 256) * c, cols // 256), :] = rows

    @pl.when(s < 16)
    def _phase0():
        gen_chunk(WA, w2_1, w2b_1, wh1_1, wh1b_1, s, _CHUNK0)
        dn0 = (((0,), (0,)), ((), ()))
        h0 = _sinpi(_f32dot(x_ref[0], w0_ref[0], dn0) + b0_ref[s])
        H[s] = h0

    @pl.when(jnp.logical_and(s >= 16, s < 16 + _PH))
    def _phase1():
        t = s - 16
        gen_chunk(WB, w2_2, w2b_2, wh1_2, wh1b_2, t, _CHUNK)
        b, p = t // 2, t % 2
        hin = H[b, pl.ds(_NHALF * p, _NHALF), :]
        h1 = _sinpi(_f32dot(hin, WA[b], _DN) + b1_ref[b])
        H[b, pl.ds(_NHALF * p, _NHALF), :] = h1

    @pl.when(jnp.logical_and(s >= 16 + _PH, s < 16 + 2 * _PH))
    def _phase2():
        t = s - (16 + _PH)
        gen_chunk(WA, w2_3, w2b_3, wh1_3, wh1b_3, t, _CHUNK)
        b, p = t // 2, t % 2
        hin = H[b, pl.ds(_NHALF * p, _NHALF), :]
        h2 = _sinpi(_f32dot(hin, WB[b], _DN) + b2_ref[b])
        H[b, pl.ds(_NHALF * p, _NHALF), :] = h2

    @pl.when(s >= 16 + 2 * _PH)
    def _phase3():
        t = s - (16 + 2 * _PH)
        h3 = _sinpi(_f32dot(H[t], WA[t], _DN) + b3_ref[t])
        out_ref[0] = _f32dot(w4_ref[t], h3, _DN) + b4_ref[t]


def kernel(query_points, z_object, hyper_params):
    hp = hyper_params
    B, Z = z_object.shape
    _, N, D = query_points.shape
    H = 256
    row = lambda v: v.reshape(1, -1)
    W0, W4, b0, b1, b2, b3, b4 = _gen_small(z_object, hp)

    full = lambda shape: pl.BlockSpec(shape, lambda s: (0,) * len(shape))
    nsteps = 32 + 2 * _PH

    def stream_spec(lo, chunk, nc):
        return pl.BlockSpec((H, chunk),
                            lambda s, lo=lo: (0, jnp.clip(s - lo, 0, nc - 1)))

    def streamb_spec(lo, chunk, nc):
        return pl.BlockSpec((1, chunk),
                            lambda s, lo=lo: (0, jnp.clip(s - lo, 0, nc - 1)))

    ex0 = lambda s: (jnp.minimum(s, 15), 0, 0)
    in_specs = [
        pl.BlockSpec((1, D, N), ex0),
        full((B, Z)),
    ]
    for l in (1, 2, 3):
        in_specs += [full((Z, H)), full((1, H))]
    in_specs += [stream_spec(0, _CHUNK0, 16), streamb_spec(0, _CHUNK0, 16),
                 stream_spec(16, _CHUNK, _PH), streamb_spec(16, _CHUNK, _PH),
                 stream_spec(16 + _PH, _CHUNK, _PH),
                 streamb_spec(16 + _PH, _CHUNK, _PH)]
    in_specs += [pl.BlockSpec((1, D, H), ex0), full((B, 1, H))]
    in_specs += [full((B, 1, H))] * 4
    in_specs += [full((B, 1, 1))]

    args = [query_points.transpose(0, 2, 1), z_object]
    for l in (1, 2, 3):
        args += [hp[f'w_h1_{l}'], row(hp[f'w_h1b_{l}'])]
    stream_args = []
    for l in (1, 2, 3):
        stream_args += [hp[f'w_h2_{l}'], row(hp[f'w_h2b_{l}'])]
    args += stream_args
    args += [W0.reshape(B, H, D).transpose(0, 2, 1), W4.reshape(B, 1, H),
             b0.reshape(B, 1, H), b1.reshape(B, 1, H),
             b2.reshape(B, 1, H), b3.reshape(B, 1, H),
             b4.reshape(B, 1, 1)]

    out = pl.pallas_call(
        _mega_body,
        grid=(nsteps,),
        in_specs=in_specs,
        out_specs=pl.BlockSpec((1, 1, N),
                               lambda s: (jnp.clip(s - (16 + 2 * _PH), 0, B - 1),
                                          0, 0)),
        out_shape=jax.ShapeDtypeStruct((B, 1, N), jnp.float32),
        scratch_shapes=[
            pltpu.VMEM((B, N, H), jnp.float32),
            pltpu.VMEM((B, H, H), jnp.float32),
            pltpu.VMEM((B, H, H), jnp.float32),
        ],
    )(*args)
    return out.reshape(B, N)
